# baseline (speedup 1.0000x reference)
import functools

import jax
import jax.numpy as jnp
from jax import lax
from jax.experimental import pallas as pl
from jax.experimental.pallas import tpu as pltpu
from jax.experimental.pallas import tpu_sc as plsc

NUM_FIELDS = 26
VOCAB = 100000
DIM = 64
BATCH = 16384

_info = plsc.get_sparse_core_info()
_NC, _NS, _L = _info.num_cores, _info.num_subcores, _info.num_lanes
_NW = _NC * _NS
_BPW = BATCH // _NW
_IPW = _BPW * NUM_FIELDS
_CB = 4
_ROWS = _CB * NUM_FIELDS
_NCHUNK = _BPW // _CB


@functools.partial(
    pl.kernel,
    mesh=plsc.VectorSubcoreMesh(core_axis_name="c", subcore_axis_name="s"),
    compiler_params=pltpu.CompilerParams(use_tc_tiling_on_sc=False),
    out_type=jax.ShapeDtypeStruct((BATCH, DIM), jnp.float32),
    scratch_types=[
        pltpu.VMEM((_IPW,), jnp.int32),
        pltpu.VMEM((_ROWS, DIM), jnp.float32),
        pltpu.VMEM((_BPW, DIM), jnp.float32),
        pltpu.SemaphoreType.DMA,
    ],
)
def _emb_sum(idx_hbm, tab_hbm, out_hbm, idx_v, rows_v, out_v, sem):
    wid = lax.axis_index("s") * _NC + lax.axis_index("c")
    base_b = wid * _BPW

    pltpu.sync_copy(idx_hbm.at[pl.ds(base_b * NUM_FIELDS, _IPW)], idx_v)
    lane = lax.iota(jnp.int32, _L)

    def _convert(i, _):
        off = i * _L
        pos = lane + off
        field = lax.rem(pos, NUM_FIELDS)
        idx_v[pl.ds(off, _L)] = idx_v[pl.ds(off, _L)] + field * VOCAB
        return _

    lax.fori_loop(0, _IPW // _L, _convert, None)

    def _chunk(c, _):
        pltpu.async_copy(
            tab_hbm.at[idx_v.at[pl.ds(c * _ROWS, _ROWS)]], rows_v, sem
        ).wait()
        for b in range(_CB):
            for j in range(DIM // _L):
                col = pl.ds(j * _L, _L)
                acc = rows_v[b * NUM_FIELDS, col]
                for f in range(1, NUM_FIELDS):
                    acc = acc + rows_v[b * NUM_FIELDS + f, col]
                out_v[c * _CB + b, col] = acc
        return _

    lax.fori_loop(0, _NCHUNK, _chunk, None)

    pltpu.sync_copy(out_v, out_hbm.at[pl.ds(base_b, _BPW)])


def kernel(inputs, tables):
    flat_idx = inputs.reshape(-1)
    flat_tab = tables.reshape(NUM_FIELDS * VOCAB, DIM)
    return _emb_sum(flat_idx, flat_tab)
